# Initial kernel scaffold; baseline (speedup 1.0000x reference)
import jax
import jax.numpy as jnp
from jax.experimental import pallas as pl


def kernel(x, adj, wild_adj, wild_feature, nodes, mutaion_site, aux, fc0_w, fc0_b, conv_w, fc_w, fc_b, fc2_w, fc2_b, fc3_w, fc3_b):
    raise NotImplementedError("write your pallas kernel here")


# speedup vs baseline: 1.2039x; 1.2039x over previous
import math

import jax
import jax.numpy as jnp
from jax.experimental import pallas as pl
from jax.experimental.pallas import tpu as pltpu

N = 4096
NFEAT = 128
NHID = 64
NLAYERS = 4
ALPHA = 0.1
LAMDA = 0.5

BLK = 512
NBLK = N // BLK


def _dot_t(a, b):
    return jax.lax.dot_general(a, b, (((1,), (1,)), ((), ())),
                               preferred_element_type=jnp.float32)


def _gcnii_kernel(adj_ref, wadj_ref, x_ref, wf_ref, mut_ref, aux_ref,
                  fc0_w_ref, fc0_b_ref, conv_w_ref,
                  fc_w_ref, fc_b_ref, fc2_w_ref, fc2_b_ref, fc3_w_ref, fc3_b_ref,
                  o_ref, gbdt_ref,
                  L_ref, WL_ref, h0_ref, wh0_ref):
    i = pl.program_id(0)
    r = pl.program_id(1)

    @pl.when(jnp.logical_and(i == 0, r == 0))
    def _prologue():
        h0 = jnp.maximum(_dot_t(x_ref[...], fc0_w_ref[...]) + fc0_b_ref[...], 0.0)
        wh0 = jnp.maximum(_dot_t(wf_ref[...], fc0_w_ref[...]) + fc0_b_ref[...], 0.0)
        h0_ref[...] = h0
        wh0_ref[...] = wh0
        L_ref[0] = h0
        WL_ref[0] = wh0

    src = jax.lax.rem(i, 2)
    dst = jax.lax.rem(i + 1, 2)
    theta = jnp.log(LAMDA / (i.astype(jnp.float32) + 1.0) + 1.0)
    w_i = conv_w_ref[i]

    rows = pl.ds(r * BLK, BLK)

    hi = jnp.dot(adj_ref[...], L_ref[src], preferred_element_type=jnp.float32)
    support = (1.0 - ALPHA) * hi + ALPHA * h0_ref[rows, :]
    out = theta * jnp.dot(support, w_i, preferred_element_type=jnp.float32) \
        + (1.0 - theta) * support
    L_ref[dst, rows, :] = jnp.maximum(out + L_ref[src, rows, :], 0.0)

    whi = jnp.dot(wadj_ref[...], WL_ref[src], preferred_element_type=jnp.float32)
    wsupport = (1.0 - ALPHA) * whi + ALPHA * wh0_ref[rows, :]
    wout = theta * jnp.dot(wsupport, w_i, preferred_element_type=jnp.float32) \
        + (1.0 - theta) * wsupport
    WL_ref[dst, rows, :] = jnp.maximum(wout + WL_ref[src, rows, :], 0.0)

    @pl.when(jnp.logical_and(i == NLAYERS - 1, r == NBLK - 1))
    def _head():
        acc_a = jnp.zeros((1, NHID), jnp.float32)
        acc_b = jnp.zeros((1, NHID), jnp.float32)
        for k in range(32):
            idx = mut_ref[k]
            acc_a = acc_a + L_ref[NLAYERS % 2, pl.ds(idx, 1), :]
            acc_b = acc_b + WL_ref[NLAYERS % 2, pl.ds(idx, 1), :]
        a = acc_a * (1.0 / 32.0)
        b = acc_b * (1.0 / 32.0)
        differ = a - b
        gbdt_ref[...] = jnp.concatenate([a, b, differ], axis=1)
        d = jnp.concatenate([jnp.maximum(differ, 0.0), aux_ref[...]], axis=1)
        o1 = jnp.maximum(_dot_t(d, fc_w_ref[...]) + fc_b_ref[...], 0.0)
        o2 = jnp.maximum(_dot_t(o1, fc2_w_ref[...]) + fc2_b_ref[...], 0.0)
        o_ref[0] = jnp.sum(o2 * fc3_w_ref[...]) + fc3_b_ref[0]


def kernel(x, adj, wild_adj, wild_feature, nodes, mutaion_site, aux,
           fc0_w, fc0_b, conv_w, fc_w, fc_b, fc2_w, fc2_b, fc3_w, fc3_b):
    del nodes

    mut2 = mutaion_site
    aux2 = aux.astype(jnp.float32).reshape(1, 57)
    fc0_b2 = fc0_b.reshape(1, NHID)
    fc_b2 = fc_b.reshape(1, NHID // 2)
    fc2_b2 = fc2_b.reshape(1, NHID // 4)
    fc3_b2 = fc3_b

    full = lambda shape: pl.BlockSpec(shape, lambda i, r: (0,) * len(shape))
    o, gbdt = pl.pallas_call(
        _gcnii_kernel,
        grid=(NLAYERS, NBLK),
        in_specs=[
            pl.BlockSpec((BLK, N), lambda i, r: (r, 0)),
            pl.BlockSpec((BLK, N), lambda i, r: (r, 0)),
            full((N, NFEAT)),
            full((N, NFEAT)),
            pl.BlockSpec(memory_space=pltpu.MemorySpace.SMEM),
            full((1, 57)),
            full((NHID, NFEAT)),
            full((1, NHID)),
            full((NLAYERS, NHID, NHID)),
            full((NHID // 2, NHID + 57)),
            full((1, NHID // 2)),
            full((NHID // 4, NHID // 2)),
            full((1, NHID // 4)),
            full((1, NHID // 4)),
            pl.BlockSpec(memory_space=pltpu.MemorySpace.SMEM),
        ],
        out_specs=[pl.BlockSpec(memory_space=pltpu.MemorySpace.SMEM),
                   full((1, 3 * NHID))],
        out_shape=[
            jax.ShapeDtypeStruct((1,), jnp.float32),
            jax.ShapeDtypeStruct((1, 3 * NHID), jnp.float32),
        ],
        scratch_shapes=[
            pltpu.VMEM((2, N, NHID), jnp.float32),
            pltpu.VMEM((2, N, NHID), jnp.float32),
            pltpu.VMEM((N, NHID), jnp.float32),
            pltpu.VMEM((N, NHID), jnp.float32),
        ],
        compiler_params=pltpu.CompilerParams(
            dimension_semantics=("arbitrary", "arbitrary"),
        ),
    )(adj, wild_adj, x, wild_feature, mut2, aux2,
      fc0_w, fc0_b2, conv_w, fc_w, fc_b2, fc2_w, fc2_b2, fc3_w, fc3_b2)
    return (o, gbdt.reshape(3 * NHID))


# speedup vs baseline: 1.2951x; 1.0758x over previous
import math

import jax
import jax.numpy as jnp
from jax.experimental import pallas as pl
from jax.experimental.pallas import tpu as pltpu

N = 4096
NFEAT = 128
NHID = 64
NLAYERS = 4
ALPHA = 0.1
LAMDA = 0.5

BLK = 256
NBLK = N // BLK
K_STREAM = 9
K_RES = NBLK - K_STREAM


def _dot_t(a, b):
    return jax.lax.dot_general(a, b, (((1,), (1,)), ((), ())),
                               preferred_element_type=jnp.float32)


def _dot(a, b):
    return jnp.dot(a, b, preferred_element_type=jnp.float32)


def _gcnii_kernel(adj_ref, wadj_ref, x_ref, wf_ref, mut_ref, aux_ref,
                  fc0_w_ref, fc0_b_ref, conv_w_ref,
                  fc_w_ref, fc_b_ref, fc2_w_ref, fc2_b_ref, fc3_w_ref, fc3_b_ref,
                  o_ref, gbdt_ref,
                  S_ref, s0_ref, adj16_ref, wadj16_ref, hi_ref):
    i = pl.program_id(0)
    r = pl.program_id(1)

    @pl.when(jnp.logical_and(i == 0, r == 0))
    def _prologue():
        h0 = jnp.maximum(_dot_t(x_ref[...], fc0_w_ref[...]) + fc0_b_ref[...], 0.0)
        wh0 = jnp.maximum(_dot_t(wf_ref[...], fc0_w_ref[...]) + fc0_b_ref[...], 0.0)
        s0_ref[:, 0:NHID] = h0
        s0_ref[:, NHID:2 * NHID] = wh0
        S_ref[0, :, 0:NHID] = h0
        S_ref[0, :, NHID:2 * NHID] = wh0

    src = jax.lax.rem(i, 2)
    dst = jax.lax.rem(i + 1, 2)
    theta = jnp.log(LAMDA / (i.astype(jnp.float32) + 1.0) + 1.0)
    w_i = conv_w_ref[i]

    rows = pl.ds(r * BLK, BLK)

    @pl.when(i == 0)
    def _first_layer():
        blk = adj_ref[...]
        wblk = wadj_ref[...]
        L = S_ref[src, :, 0:NHID]
        WL = S_ref[src, :, NHID:2 * NHID]
        hi_ref[:, 0:NHID] = _dot(blk, L)
        hi_ref[:, NHID:2 * NHID] = _dot(wblk, WL)

        @pl.when(r >= K_STREAM)
        def _park():
            res = pl.ds((r - K_STREAM) * BLK, BLK)
            adj16_ref[res, :] = blk.astype(jnp.bfloat16)
            wadj16_ref[res, :] = wblk.astype(jnp.bfloat16)

    @pl.when(i > 0)
    def _later_layers():
        L16 = S_ref[src, :, 0:NHID].astype(jnp.bfloat16)
        WL16 = S_ref[src, :, NHID:2 * NHID].astype(jnp.bfloat16)

        @pl.when(r < K_STREAM)
        def _streamed():
            L = S_ref[src, :, 0:NHID]
            WL = S_ref[src, :, NHID:2 * NHID]
            hi_ref[:, 0:NHID] = _dot(adj_ref[...], L)
            hi_ref[:, NHID:2 * NHID] = _dot(wadj_ref[...], WL)

        @pl.when(r >= K_STREAM)
        def _resident():
            res = pl.ds((r - K_STREAM) * BLK, BLK)
            hi_ref[:, 0:NHID] = _dot(adj16_ref[res, :], L16)
            hi_ref[:, NHID:2 * NHID] = _dot(wadj16_ref[res, :], WL16)

    hi = hi_ref[...]
    support = (1.0 - ALPHA) * hi + ALPHA * s0_ref[rows, :]
    conv = theta * jnp.concatenate(
        [_dot(support[:, 0:NHID], w_i), _dot(support[:, NHID:2 * NHID], w_i)],
        axis=1)
    out = conv + (1.0 - theta) * support
    S_ref[dst, rows, :] = jnp.maximum(out + S_ref[src, rows, :], 0.0)

    @pl.when(jnp.logical_and(i == NLAYERS - 1, r == NBLK - 1))
    def _head():
        acc = jnp.zeros((1, 2 * NHID), jnp.float32)
        for k in range(32):
            idx = mut_ref[k]
            acc = acc + S_ref[NLAYERS % 2, pl.ds(idx, 1), :]
        a = acc[:, 0:NHID] * (1.0 / 32.0)
        b = acc[:, NHID:2 * NHID] * (1.0 / 32.0)
        differ = a - b
        gbdt_ref[...] = jnp.concatenate([a, b, differ], axis=1)
        d = jnp.concatenate([jnp.maximum(differ, 0.0), aux_ref[...]], axis=1)
        o1 = jnp.maximum(_dot_t(d, fc_w_ref[...]) + fc_b_ref[...], 0.0)
        o2 = jnp.maximum(_dot_t(o1, fc2_w_ref[...]) + fc2_b_ref[...], 0.0)
        o_ref[0] = jnp.sum(o2 * fc3_w_ref[...]) + fc3_b_ref[0]


def kernel(x, adj, wild_adj, wild_feature, nodes, mutaion_site, aux,
           fc0_w, fc0_b, conv_w, fc_w, fc_b, fc2_w, fc2_b, fc3_w, fc3_b):
    del nodes

    aux2 = aux.astype(jnp.float32).reshape(1, 57)
    fc0_b2 = fc0_b.reshape(1, NHID)
    fc_b2 = fc_b.reshape(1, NHID // 2)
    fc2_b2 = fc2_b.reshape(1, NHID // 4)

    def adj_map(i, r):
        return (jnp.where((i == 0) | (r < K_STREAM), r, K_STREAM - 1), 0)

    full = lambda shape: pl.BlockSpec(shape, lambda i, r: (0,) * len(shape))
    o, gbdt = pl.pallas_call(
        _gcnii_kernel,
        grid=(NLAYERS, NBLK),
        in_specs=[
            pl.BlockSpec((BLK, N), adj_map),
            pl.BlockSpec((BLK, N), adj_map),
            full((N, NFEAT)),
            full((N, NFEAT)),
            pl.BlockSpec(memory_space=pltpu.MemorySpace.SMEM),
            full((1, 57)),
            full((NHID, NFEAT)),
            full((1, NHID)),
            full((NLAYERS, NHID, NHID)),
            full((NHID // 2, NHID + 57)),
            full((1, NHID // 2)),
            full((NHID // 4, NHID // 2)),
            full((1, NHID // 4)),
            full((1, NHID // 4)),
            pl.BlockSpec(memory_space=pltpu.MemorySpace.SMEM),
        ],
        out_specs=[pl.BlockSpec(memory_space=pltpu.MemorySpace.SMEM),
                   full((1, 3 * NHID))],
        out_shape=[
            jax.ShapeDtypeStruct((1,), jnp.float32),
            jax.ShapeDtypeStruct((1, 3 * NHID), jnp.float32),
        ],
        scratch_shapes=[
            pltpu.VMEM((2, N, 2 * NHID), jnp.float32),
            pltpu.VMEM((N, 2 * NHID), jnp.float32),
            pltpu.VMEM((K_RES * BLK, N), jnp.bfloat16),
            pltpu.VMEM((K_RES * BLK, N), jnp.bfloat16),
            pltpu.VMEM((BLK, 2 * NHID), jnp.float32),
        ],
        compiler_params=pltpu.CompilerParams(
            dimension_semantics=("arbitrary", "arbitrary"),
            vmem_limit_bytes=67_000_000,
        ),
    )(adj, wild_adj, x, wild_feature, mutaion_site, aux2,
      fc0_w, fc0_b2, conv_w, fc_w, fc_b2, fc2_w, fc2_b2, fc3_w, fc3_b)
    return (o, gbdt.reshape(3 * NHID))
